# baseline (speedup 1.0000x reference)
import jax
import jax.numpy as jnp
from jax import lax
from jax.experimental import pallas as pl
from jax.experimental.pallas import tpu as pltpu
from jax.experimental.pallas import tpu_sc as plsc

COUNT_DIM = 100
N_EMBD = 64
BATCH = 16384
NVALS = 100

L = 16
NC = 2
NS = 16
NW = NC * NS
CB = 256
NCHUNK = BATCH // CB
ITEMS = COUNT_DIM * NCHUNK
IPW = ITEMS // NW
CNT_PER_W = IPW * CB
TSTRIDE = N_EMBD + 1


def _sc_body(cnt_hbm, val_hbm, bit_hbm, out_hbm,
             val_v, bit_v, tab_v, cnt_v, ob0, ob1, sem0, sem1):
    wid = lax.axis_index("s") * NC + lax.axis_index("c")

    pltpu.sync_copy(val_hbm, val_v)
    pltpu.sync_copy(bit_hbm, bit_v)
    pltpu.sync_copy(cnt_hbm.at[pl.ds(wid * CNT_PER_W, CNT_PER_W)], cnt_v)

    def do_item(t, ob, sem, first):
        item = wid * IPW + t
        d = item // NCHUNK
        ch = item - d * NCHUNK
        b0 = ch * CB

        @pl.when(jnp.logical_or(ch == 0, t == 0))
        def _build():
            brow = [bit_v[pl.ds(d * N_EMBD + L * j, L)] for j in range(4)]

            @plsc.parallel_loop(0, NVALS, unroll=4)
            def build_body(v):
                for j in range(4):
                    tab_v[pl.ds(v * TSTRIDE + L * j, L)] = (
                        val_v[pl.ds(v * N_EMBD + L * j, L)] + brow[j])

        @pl.when(jnp.logical_not(first))
        def _drain():
            pltpu.make_async_copy(ob, out_hbm.at[0, :, pl.ds(0, CB)], sem).wait()

        cblk = t * CB

        @plsc.parallel_loop(0, CB // L, unroll=2)
        def group_body(g):
            cbase = cnt_v[pl.ds(cblk + g * L, L)] * TSTRIDE
            for e in range(N_EMBD):
                ob[e, pl.ds(g * L, L)] = plsc.load_gather(tab_v, [cbase + e])

        pltpu.async_copy(ob, out_hbm.at[d, :, pl.ds(b0, CB)], sem)

    def pair_body(t2, carry):
        do_item(2 * t2, ob0, sem0, t2 == 0)
        do_item(2 * t2 + 1, ob1, sem1, t2 == 0)
        return carry

    lax.fori_loop(0, IPW // 2, pair_body, 0, unroll=False)

    pltpu.make_async_copy(ob0, out_hbm.at[0, :, pl.ds(0, CB)], sem0).wait()
    pltpu.make_async_copy(ob1, out_hbm.at[0, :, pl.ds(0, CB)], sem1).wait()


def kernel(count, val_emb, bit_emb):
    cnt_flat = count.astype(jnp.int32).T.reshape(-1)
    val_flat = val_emb.reshape(-1)
    bit_flat = bit_emb.reshape(-1)

    mesh = plsc.VectorSubcoreMesh(core_axis_name="c", subcore_axis_name="s")
    f = pl.kernel(
        _sc_body,
        mesh=mesh,
        compiler_params=pltpu.CompilerParams(needs_layout_passes=False),
        out_type=jax.ShapeDtypeStruct((COUNT_DIM, N_EMBD, BATCH), jnp.float32),
        scratch_types=[
            pltpu.VMEM((NVALS * N_EMBD,), jnp.float32),
            pltpu.VMEM((COUNT_DIM * N_EMBD,), jnp.float32),
            pltpu.VMEM((NVALS * TSTRIDE,), jnp.float32),
            pltpu.VMEM((CNT_PER_W,), jnp.int32),
            pltpu.VMEM((N_EMBD, CB), jnp.float32),
            pltpu.VMEM((N_EMBD, CB), jnp.float32),
            pltpu.SemaphoreType.DMA,
            pltpu.SemaphoreType.DMA,
        ],
    )
    out3 = f(cnt_flat, val_flat, bit_flat)
    return jnp.transpose(out3, (2, 0, 1))


# speedup vs baseline: 2.2295x; 2.2295x over previous
import jax
import jax.numpy as jnp
from jax import lax
from jax.experimental import pallas as pl
from jax.experimental.pallas import tpu as pltpu
from jax.experimental.pallas import tpu_sc as plsc

COUNT_DIM = 100
N_EMBD = 64
BATCH = 16384
NVALS = 100

L = 16
NC = 2
NS = 16
NW = NC * NS
CB = 512
NCHUNK = BATCH // CB
ITEMS = COUNT_DIM * NCHUNK
IPW = ITEMS // NW
TSTRIDE = N_EMBD + 1


def _sc_body(cntT_hbm, val_hbm, bit_hbm, out_hbm,
             val_v, bit_v, tab_v, cb0, cb1, ob0, ob1,
             sem0, sem1, csem0, csem1):
    wid = lax.axis_index("s") * NC + lax.axis_index("c")

    pltpu.sync_copy(val_hbm, val_v)
    pltpu.sync_copy(bit_hbm, bit_v)

    def cnt_slice(t):
        item = wid * IPW + t
        d = item // NCHUNK
        b0 = (item - d * NCHUNK) * CB
        return cntT_hbm.at[d, pl.ds(b0, CB)]

    pltpu.async_copy(cnt_slice(0), cb0, csem0)

    def do_item(t, cnt_v, csem, pf_t, pf_cb, pf_csem, ob, sem, first):
        item = wid * IPW + t
        d = item // NCHUNK
        ch = item - d * NCHUNK
        b0 = ch * CB

        pltpu.make_async_copy(cnt_slice(t), cnt_v, csem).wait()
        pltpu.async_copy(cnt_slice(pf_t), pf_cb, pf_csem)

        @pl.when(jnp.logical_or(ch == 0, t == 0))
        def _build():
            brow = [bit_v[pl.ds(d * N_EMBD + L * j, L)] for j in range(4)]

            @plsc.parallel_loop(0, NVALS, unroll=4)
            def build_body(v):
                for j in range(4):
                    tab_v[pl.ds(v * TSTRIDE + L * j, L)] = (
                        val_v[pl.ds(v * N_EMBD + L * j, L)] + brow[j])

        @pl.when(jnp.logical_not(first))
        def _drain():
            pltpu.make_async_copy(ob, out_hbm.at[0, :, pl.ds(0, CB)], sem).wait()

        @plsc.parallel_loop(0, CB // L, unroll=2)
        def group_body(g):
            cbase = cnt_v[pl.ds(g * L, L)] * TSTRIDE
            for e in range(N_EMBD):
                ob[e, pl.ds(g * L, L)] = plsc.load_gather(tab_v, [cbase + e])

        pltpu.async_copy(ob, out_hbm.at[d, :, pl.ds(b0, CB)], sem)

    def pair_body(t2, carry):
        ta = 2 * t2
        tb = 2 * t2 + 1
        tb_pf = jnp.minimum(tb + 1, IPW - 1)
        do_item(ta, cb0, csem0, tb, cb1, csem1, ob0, sem0, t2 == 0)
        do_item(tb, cb1, csem1, tb_pf, cb0, csem0, ob1, sem1, t2 == 0)
        return carry

    lax.fori_loop(0, IPW // 2, pair_body, 0, unroll=False)

    pltpu.make_async_copy(cnt_slice(0), cb0, csem0).wait()
    pltpu.make_async_copy(ob0, out_hbm.at[0, :, pl.ds(0, CB)], sem0).wait()
    pltpu.make_async_copy(ob1, out_hbm.at[0, :, pl.ds(0, CB)], sem1).wait()


def kernel(count, val_emb, bit_emb):
    cnt_t = count.astype(jnp.int32).T
    val_flat = val_emb.reshape(-1)
    bit_flat = bit_emb.reshape(-1)

    mesh = plsc.VectorSubcoreMesh(core_axis_name="c", subcore_axis_name="s")
    f = pl.kernel(
        _sc_body,
        mesh=mesh,
        compiler_params=pltpu.CompilerParams(needs_layout_passes=False),
        out_type=jax.ShapeDtypeStruct((COUNT_DIM, N_EMBD, BATCH), jnp.float32),
        scratch_types=[
            pltpu.VMEM((NVALS * N_EMBD,), jnp.float32),
            pltpu.VMEM((COUNT_DIM * N_EMBD,), jnp.float32),
            pltpu.VMEM((NVALS * TSTRIDE,), jnp.float32),
            pltpu.VMEM((CB,), jnp.int32),
            pltpu.VMEM((CB,), jnp.int32),
            pltpu.VMEM((N_EMBD, CB), jnp.float32),
            pltpu.VMEM((N_EMBD, CB), jnp.float32),
            pltpu.SemaphoreType.DMA,
            pltpu.SemaphoreType.DMA,
            pltpu.SemaphoreType.DMA,
            pltpu.SemaphoreType.DMA,
        ],
    )
    out3 = f(cnt_t, val_flat, bit_flat)
    return jnp.transpose(out3, (2, 0, 1))
